# baseline (speedup 1.0000x reference)
import dataclasses
import functools

import jax
import jax.numpy as jnp
from jax import lax
from jax.experimental import pallas as pl
from jax.experimental.pallas import tpu as pltpu
from jax.experimental.pallas import tpu_sc as plsc

NUM = 1000000
DIM = 32
BATCH = 16384

NC = 2
NS = 16
L = 16
NW = NC * NS
BPW = BATCH // NW
CHUNK = 128
NCHUNK = BPW // CHUNK
PACK = 128 // DIM
NROW = NUM // PACK


def _compiler_params():
    cp = pltpu.CompilerParams()
    fields = pltpu.CompilerParams.__dataclass_fields__
    if "needs_layout_passes" in fields:
        cp = dataclasses.replace(cp, needs_layout_passes=False)
    return cp


def _dot_kernel(aidx_hbm, bidx_hbm, a4_hbm, b4_hbm, out_hbm,
                ai_v, bi_v, ar_v, br_v, o_v, sem_a, sem_b):
    wid = lax.axis_index("s") * NC + lax.axis_index("c")
    base = wid * BPW

    pltpu.sync_copy(aidx_hbm.at[wid], ai_v)
    pltpu.sync_copy(bidx_hbm.at[wid], bi_v)

    lane = lax.iota(jnp.int32, 16)

    def fire(j):
        buf = j % 2
        ca = pltpu.async_copy(a4_hbm.at[ai_v.at[j]], ar_v.at[buf], sem_a)
        cb = pltpu.async_copy(b4_hbm.at[bi_v.at[j]], br_v.at[buf], sem_b)
        return ca, cb

    def compute(j):
        buf = j % 2

        @pl.loop(0, CHUNK, step=L)
        def _(g):
            rows = lane + g
            soff_a = ai_v[NCHUNK + j, pl.ds(g, L)]
            soff_b = bi_v[NCHUNK + j, pl.ds(g, L)]
            acc = None
            for d in range(DIM):
                av = plsc.load_gather(ar_v.at[buf], [rows, soff_a + d])
                bv = plsc.load_gather(br_v.at[buf], [rows, soff_b + d])
                prod = av * bv
                acc = prod if acc is None else acc + prod
            o_v[pl.ds(j * CHUNK + g, L)] = acc

    pending = fire(0)
    for j in range(NCHUNK):
        nxt = fire(j + 1) if j + 1 < NCHUNK else None
        pending[0].wait()
        pending[1].wait()
        compute(j)
        pending = nxt

    pltpu.sync_copy(o_v, out_hbm.at[pl.ds(base, BPW)])


def _stage_indices(idx):
    blk = (idx // PACK).reshape(NW, NCHUNK, CHUNK)
    off = ((idx % PACK) * DIM).reshape(NW, NCHUNK, CHUNK)
    return jnp.concatenate([blk, off], axis=1)


@jax.jit
def kernel(aIdx, bIdx, A, B):
    aIdx = aIdx.astype(jnp.int32)
    bIdx = bIdx.astype(jnp.int32)
    one = jnp.where(aIdx[0] >= 0, jnp.float32(1.0), jnp.float32(2.0))
    a4 = A.reshape(NROW, 128) * one
    b4 = B.reshape(NROW, 128) * one
    ai = _stage_indices(aIdx)
    bi = _stage_indices(bIdx)
    mesh = plsc.VectorSubcoreMesh(core_axis_name="c", subcore_axis_name="s")
    run = functools.partial(
        pl.kernel,
        mesh=mesh,
        out_type=jax.ShapeDtypeStruct((BATCH,), jnp.float32),
        scratch_types=[
            pltpu.VMEM((2 * NCHUNK, CHUNK), jnp.int32),
            pltpu.VMEM((2 * NCHUNK, CHUNK), jnp.int32),
            pltpu.VMEM((2, CHUNK, 128), jnp.float32),
            pltpu.VMEM((2, CHUNK, 128), jnp.float32),
            pltpu.VMEM((BPW,), jnp.float32),
            pltpu.SemaphoreType.DMA,
            pltpu.SemaphoreType.DMA,
        ],
        compiler_params=_compiler_params(),
    )(_dot_kernel)
    return run(ai, bi, a4, b4)


# speedup vs baseline: 6.0783x; 6.0783x over previous
import dataclasses
import functools

import jax
import jax.numpy as jnp
from jax import lax
from jax.experimental import pallas as pl
from jax.experimental.pallas import tpu as pltpu
from jax.experimental.pallas import tpu_sc as plsc

NUM = 1000000
DIM = 32
BATCH = 16384

NC = 2
NS = 16
L = 16
NW = NC * NS
BPW = BATCH // NW
NSLOT = 8
HALF = NSLOT // 2


def _compiler_params():
    cp = pltpu.CompilerParams()
    fields = pltpu.CompilerParams.__dataclass_fields__
    if "needs_layout_passes" in fields:
        cp = dataclasses.replace(cp, needs_layout_passes=False)
    return cp


def _dot_kernel(aidx_hbm, bidx_hbm, at_hbm, bt_hbm, out_hbm,
                ai_s, bi_s, a_slots, b_slots, o2_v, o_v,
                sem_a0, sem_a1, sem_b0, sem_b1):
    wid = lax.axis_index("s") * NC + lax.axis_index("c")
    base = wid * BPW

    pltpu.sync_copy(aidx_hbm.at[pl.ds(base, BPW)], ai_s)
    pltpu.sync_copy(bidx_hbm.at[pl.ds(base, BPW)], bi_s)

    lane = lax.iota(jnp.int32, 16)
    sems = ((sem_a0, sem_b0), (sem_a1, sem_b1))

    def load_idx(p8):
        off = pl.multiple_of(p8, 8)
        return ai_s[pl.ds(off, L)], bi_s[pl.ds(off, L)]

    def fire_half(h, va, vb, e0):
        sem_a, sem_b = sems[h]
        for b in range(HALF):
            s = h * HALF + b
            ia = va[e0 + b]
            ib = vb[e0 + b]
            coff_a = pl.multiple_of((ia >> 7) * 128, 128)
            coff_b = pl.multiple_of((ib >> 7) * 128, 128)
            dst = pl.ds(s * DIM, DIM)
            pltpu.async_copy(at_hbm.at[:, pl.ds(coff_a, 128)],
                             a_slots.at[dst], sem_a)
            pltpu.async_copy(bt_hbm.at[:, pl.ds(coff_b, 128)],
                             b_slots.at[dst], sem_b)

    def drain_half(h):
        sem_a, sem_b = sems[h]
        for b in range(HALF):
            s = h * HALF + b
            dst = pl.ds(s * DIM, DIM)
            pltpu.make_async_copy(at_hbm.at[:, pl.ds(0, 128)],
                                  a_slots.at[dst], sem_a).wait()
            pltpu.make_async_copy(bt_hbm.at[:, pl.ds(0, 128)],
                                  b_slots.at[dst], sem_b).wait()

    def compute_half(h, va, vb, e0, p0):
        for b in range(HALF):
            s = h * HALF + b
            col_a = jnp.full((L,), va[e0 + b] & 127, jnp.int32)
            col_b = jnp.full((L,), vb[e0 + b] & 127, jnp.int32)
            rows_lo = lane + (s * DIM)
            rows_hi = rows_lo + 16
            a0 = plsc.load_gather(a_slots, [rows_lo, col_a])
            a1 = plsc.load_gather(a_slots, [rows_hi, col_a])
            b0 = plsc.load_gather(b_slots, [rows_lo, col_b])
            b1 = plsc.load_gather(b_slots, [rows_hi, col_b])
            off = pl.multiple_of((p0 + b) * L, 8)
            o2_v[pl.ds(off, L)] = a0 * b0 + a1 * b1

    va0, vb0 = load_idx(0)
    fire_half(0, va0, vb0, 0)
    fire_half(1, va0, vb0, HALF)

    @pl.loop(0, BPW - NSLOT, step=NSLOT)
    def _(j0):
        vc_a, vc_b = load_idx(j0)
        vn_a, vn_b = load_idx(j0 + NSLOT)
        for h in range(2):
            drain_half(h)
            compute_half(h, vc_a, vc_b, h * HALF, j0 + h * HALF)
            fire_half(h, vn_a, vn_b, h * HALF)

    vt_a, vt_b = load_idx(BPW - NSLOT)
    for h in range(2):
        drain_half(h)
        compute_half(h, vt_a, vt_b, h * HALF, (BPW - NSLOT) + h * HALF)

    @pl.loop(0, BPW, step=L)
    def _(g):
        rows16 = (lane + g) * L
        acc = None
        for k in range(L):
            v = plsc.load_gather(o2_v, [rows16 + k])
            acc = v if acc is None else acc + v
        o_v[pl.ds(g, L)] = acc

    pltpu.sync_copy(o_v, out_hbm.at[pl.ds(base, BPW)])


@jax.jit
def kernel(aIdx, bIdx, A, B):
    aIdx = aIdx.astype(jnp.int32)
    bIdx = bIdx.astype(jnp.int32)
    At = A.T
    Bt = B.T
    mesh = plsc.VectorSubcoreMesh(core_axis_name="c", subcore_axis_name="s")
    run = functools.partial(
        pl.kernel,
        mesh=mesh,
        out_type=jax.ShapeDtypeStruct((BATCH,), jnp.float32),
        scratch_types=[
            pltpu.VMEM((BPW,), jnp.int32),
            pltpu.VMEM((BPW,), jnp.int32),
            pltpu.VMEM((NSLOT * DIM, 128), jnp.float32),
            pltpu.VMEM((NSLOT * DIM, 128), jnp.float32),
            pltpu.VMEM((BPW * L,), jnp.float32),
            pltpu.VMEM((BPW,), jnp.float32),
            pltpu.SemaphoreType.DMA,
            pltpu.SemaphoreType.DMA,
            pltpu.SemaphoreType.DMA,
            pltpu.SemaphoreType.DMA,
        ],
        compiler_params=_compiler_params(),
    )(_dot_kernel)
    return run(aIdx, bIdx, At, Bt)


# speedup vs baseline: 6.0820x; 1.0006x over previous
import dataclasses
import functools

import jax
import jax.numpy as jnp
from jax import lax
from jax.experimental import pallas as pl
from jax.experimental.pallas import tpu as pltpu
from jax.experimental.pallas import tpu_sc as plsc

NUM = 1000000
DIM = 32
BATCH = 16384

NC = 2
NS = 16
L = 16
NW = NC * NS
BPW = BATCH // NW
NSLOT = 8
HALF = NSLOT // 2


def _compiler_params():
    cp = pltpu.CompilerParams()
    fields = pltpu.CompilerParams.__dataclass_fields__
    if "needs_layout_passes" in fields:
        cp = dataclasses.replace(cp, needs_layout_passes=False)
    return cp


def _dot_kernel(aidx_hbm, bidx_hbm, at_hbm, bt_hbm, out_hbm,
                ai_s, bi_s, a_slots, b_slots, o2_v, o_v,
                sem_a0, sem_a1, sem_b0, sem_b1):
    wid = lax.axis_index("s") * NC + lax.axis_index("c")
    base = wid * BPW

    pltpu.sync_copy(aidx_hbm.at[pl.ds(base, BPW)], ai_s.at[pl.ds(0, BPW)])
    pltpu.sync_copy(bidx_hbm.at[pl.ds(base, BPW)], bi_s.at[pl.ds(0, BPW)])

    lane = lax.iota(jnp.int32, 16)
    sems = ((sem_a0, sem_b0), (sem_a1, sem_b1))

    def load_idx(p8):
        off = pl.multiple_of(p8, 8)
        return ai_s[pl.ds(off, L)], bi_s[pl.ds(off, L)]

    def fire_half(h, va, vb, e0):
        sem_a, sem_b = sems[h]
        for b in range(HALF):
            s = h * HALF + b
            ia = va[e0 + b]
            ib = vb[e0 + b]
            coff_a = pl.multiple_of((ia >> 7) * 128, 128)
            coff_b = pl.multiple_of((ib >> 7) * 128, 128)
            dst = pl.ds(s * DIM, DIM)
            pltpu.async_copy(at_hbm.at[:, pl.ds(coff_a, 128)],
                             a_slots.at[dst], sem_a)
            pltpu.async_copy(bt_hbm.at[:, pl.ds(coff_b, 128)],
                             b_slots.at[dst], sem_b)

    def drain_half(h):
        sem_a, sem_b = sems[h]
        for b in range(HALF):
            s = h * HALF + b
            dst = pl.ds(s * DIM, DIM)
            pltpu.make_async_copy(at_hbm.at[:, pl.ds(0, 128)],
                                  a_slots.at[dst], sem_a).wait()
            pltpu.make_async_copy(bt_hbm.at[:, pl.ds(0, 128)],
                                  b_slots.at[dst], sem_b).wait()

    def compute_half(h, va, vb, e0, p0):
        for b in range(HALF):
            s = h * HALF + b
            col_a = jnp.full((L,), va[e0 + b] & 127, jnp.int32)
            col_b = jnp.full((L,), vb[e0 + b] & 127, jnp.int32)
            rows_lo = lane + (s * DIM)
            rows_hi = rows_lo + 16
            a0 = plsc.load_gather(a_slots, [rows_lo, col_a])
            a1 = plsc.load_gather(a_slots, [rows_hi, col_a])
            b0 = plsc.load_gather(b_slots, [rows_lo, col_b])
            b1 = plsc.load_gather(b_slots, [rows_hi, col_b])
            off = pl.multiple_of((p0 + b) * L, 8)
            o2_v[pl.ds(off, L)] = a0 * b0 + a1 * b1

    va0, vb0 = load_idx(0)
    fire_half(0, va0, vb0, 0)
    fire_half(1, va0, vb0, HALF)

    @pl.loop(0, BPW - NSLOT, step=NSLOT)
    def _(j0):
        vc_a, vc_b = load_idx(j0)
        vn_a, vn_b = load_idx(j0 + NSLOT)
        for h in range(2):
            drain_half(h)
            compute_half(h, vc_a, vc_b, h * HALF, j0 + h * HALF)
            fire_half(h, vn_a, vn_b, h * HALF)

    vt_a, vt_b = load_idx(BPW - NSLOT)
    for h in range(2):
        drain_half(h)
        compute_half(h, vt_a, vt_b, h * HALF, (BPW - NSLOT) + h * HALF)

    @pl.loop(0, BPW, step=L)
    def _(g):
        rows16 = (lane + g) * L
        acc = None
        for k in range(L):
            v = plsc.load_gather(o2_v, [rows16 + k])
            acc = v if acc is None else acc + v
        o_v[pl.ds(g, L)] = acc

    pltpu.sync_copy(o_v, out_hbm.at[pl.ds(base, BPW)])


@jax.jit
def kernel(aIdx, bIdx, A, B):
    aIdx = aIdx.astype(jnp.int32)
    bIdx = bIdx.astype(jnp.int32)
    At = A.T
    Bt = B.T
    mesh = plsc.VectorSubcoreMesh(core_axis_name="c", subcore_axis_name="s")
    run = functools.partial(
        pl.kernel,
        mesh=mesh,
        out_type=jax.ShapeDtypeStruct((BATCH,), jnp.float32),
        scratch_types=[
            pltpu.VMEM((BPW + L,), jnp.int32),
            pltpu.VMEM((BPW + L,), jnp.int32),
            pltpu.VMEM((NSLOT * DIM, 128), jnp.float32),
            pltpu.VMEM((NSLOT * DIM, 128), jnp.float32),
            pltpu.VMEM((BPW * L,), jnp.float32),
            pltpu.VMEM((BPW,), jnp.float32),
            pltpu.SemaphoreType.DMA,
            pltpu.SemaphoreType.DMA,
            pltpu.SemaphoreType.DMA,
            pltpu.SemaphoreType.DMA,
        ],
        compiler_params=_compiler_params(),
    )(_dot_kernel)
    return run(aIdx, bIdx, At, Bt)
